# Initial kernel scaffold; baseline (speedup 1.0000x reference)
import jax
import jax.numpy as jnp
from jax.experimental import pallas as pl


def kernel(inputs, class_embed, pos_table):
    raise NotImplementedError("write your pallas kernel here")


# speedup vs baseline: 1.0158x; 1.0158x over previous
import jax
import jax.numpy as jnp
from jax.experimental import pallas as pl


def _body(in_ref, cls_ref, pos_ref, out_ref):
    out_ref[0, 0, :] = cls_ref[0, 0, :] + pos_ref[0, :]
    out_ref[0, 1:, :] = in_ref[0, :, :] + pos_ref[1:, :]


def kernel(inputs, class_embed, pos_table):
    b, n_patches, d = inputs.shape
    n_tot = n_patches + 1
    return pl.pallas_call(
        _body,
        grid=(b,),
        in_specs=[
            pl.BlockSpec((1, n_patches, d), lambda i: (i, 0, 0)),
            pl.BlockSpec((1, 1, d), lambda i: (0, 0, 0)),
            pl.BlockSpec((n_tot, d), lambda i: (0, 0)),
        ],
        out_specs=pl.BlockSpec((1, n_tot, d), lambda i: (i, 0, 0)),
        out_shape=jax.ShapeDtypeStruct((b, n_tot, d), inputs.dtype),
    )(inputs, class_embed, pos_table)
